# baseline (speedup 1.0000x reference)
import jax
import jax.numpy as jnp
from jax import lax
from jax.experimental import pallas as pl
from jax.experimental.pallas import tpu as pltpu
from jax.experimental.pallas import tpu_sc as plsc

NN = 50000
EE = 800000
GW = 96
GPT = 528
EP = 16 * GPT * GW
EPR = EP // GW
OUTER = GPT // 8
NSPLIT = 4
QN = 12544
ACC_R = 12672
ZROWS = ACC_R // 16
WA = 488
WB = 176
WC = 128
BLK = 1000
GRID = NN // BLK

f32 = jnp.float32
i32 = jnp.int32


def _sc_mesh():
    return plsc.VectorSubcoreMesh(core_axis_name="c", subcore_axis_name="s")


def _spmm_body(gm, ga, srcm, dstm, srca, dsta, zrows, om, oa,
               acc, src_v, dst_v, dstq_v, rows0, rows1, sg0, sg1, ss0, ss1):
    ci = lax.axis_index("c")
    si = lax.axis_index("s")
    rows = [rows0, rows1]
    sgs = [sg0, sg1]
    sss = [ss0, ss1]

    for q in range(NSPLIT):
        lb = pl.multiple_of(si * ZROWS, 8)
        pltpu.sync_copy(zrows, acc.at[pl.ds(lb, ZROWS), :])
        plsc.subcore_barrier()

        def edge_pass(src2d, dst2d, g):
            def body(ol, carry):
                off = pl.multiple_of(si * GPT + ol * 8, 8)
                pltpu.sync_copy(src2d.at[pl.ds(off, 8)], src_v)
                pltpu.sync_copy(dst2d.at[pl.ds(off, 8)], dst_v)
                for j in range(8):
                    for v in range(6):
                        d = dst_v[j, pl.ds(16 * v, 16)]
                        ok = (d >= q * QN) & (d < q * QN + QN)
                        dstq_v[j, pl.ds(16 * v, 16)] = \
                            jnp.where(ok, d - q * QN, QN)
                scp = [None, None]
                for j in range(8):
                    b = j % 2
                    if scp[b] is not None:
                        scp[b].wait()
                    pltpu.async_copy(g.at[src_v.at[j]], rows[b],
                                     sgs[b]).wait()
                    scp[b] = pltpu.async_copy(rows[b], acc.at[dstq_v.at[j]],
                                              sss[b], add=True)
                scp[0].wait()
                scp[1].wait()
                return carry
            lax.fori_loop(0, OUTER, body, 0)

        @pl.when(ci == 0)
        def _():
            edge_pass(srcm, dstm, gm)

        @pl.when(ci == 1)
        def _():
            edge_pass(srca, dsta, ga)

        plsc.subcore_barrier()

        def wout(o):
            g0 = pl.multiple_of(q * QN + si * ZROWS, 8)
            pltpu.sync_copy(acc.at[pl.ds(lb, WA), :], o.at[pl.ds(g0, WA), :])

            def wb():
                pltpu.sync_copy(acc.at[pl.ds(lb + WA, WB), :],
                                o.at[pl.ds(g0 + WA, WB), :])
            if q < NSPLIT - 1:
                wb()
            else:
                @pl.when(si < 15)
                def _():
                    wb()

            @pl.when(si < 15)
            def _():
                pltpu.sync_copy(acc.at[pl.ds(lb + WA + WB, WC), :],
                                o.at[pl.ds(g0 + WA + WB, WC), :])

        @pl.when(ci == 0)
        def _():
            wout(om)

        @pl.when(ci == 1)
        def _():
            wout(oa)


def _spmm_call(gm, ga, srcm, dstm, srca, dsta, zrows):
    return pl.kernel(
        _spmm_body,
        out_type=[jax.ShapeDtypeStruct((NN, 128), f32)] * 2,
        mesh=_sc_mesh(),
        scratch_types=[
            pltpu.VMEM_SHARED((ACC_R, 128), f32),
            pltpu.VMEM((8, GW), i32),
            pltpu.VMEM((8, GW), i32),
            pltpu.VMEM((8, GW), i32),
            pltpu.VMEM((GW, 128), f32),
            pltpu.VMEM((GW, 128), f32),
            pltpu.SemaphoreType.DMA,
            pltpu.SemaphoreType.DMA,
            pltpu.SemaphoreType.DMA,
            pltpu.SemaphoreType.DMA,
        ],
    )(gm, ga, srcm, dstm, srca, dsta, zrows)


def _degs_body(dstm, dsta, ones128, zrows, om, oa, acc, dst_v, dstq_v, ones_v,
               sd0):
    ci = lax.axis_index("c")
    si = lax.axis_index("s")
    pltpu.sync_copy(ones128, ones_v)

    for q in range(NSPLIT):
        lb = pl.multiple_of(si * ZROWS, 8)
        pltpu.sync_copy(zrows, acc.at[pl.ds(lb, ZROWS), :])
        plsc.subcore_barrier()

        def edge_pass(dst2d):
            def body(ol, carry):
                off = pl.multiple_of(si * GPT + ol * 8, 8)
                pltpu.sync_copy(dst2d.at[pl.ds(off, 8)], dst_v)
                for j in range(8):
                    for v in range(6):
                        d = dst_v[j, pl.ds(16 * v, 16)]
                        ok = (d >= q * QN) & (d < q * QN + QN)
                        dstq_v[j, pl.ds(16 * v, 16)] = \
                            jnp.where(ok, d - q * QN, QN)
                cps = [pltpu.async_copy(ones_v, acc.at[dstq_v.at[j]],
                                        sd0, add=True) for j in range(8)]
                for cp in cps:
                    cp.wait()
                return carry
            lax.fori_loop(0, OUTER, body, 0)

        @pl.when(ci == 0)
        def _():
            edge_pass(dstm)

        @pl.when(ci == 1)
        def _():
            edge_pass(dsta)

        plsc.subcore_barrier()

        def wout(o):
            g0 = pl.multiple_of(q * QN + si * ZROWS, 8)
            pltpu.sync_copy(acc.at[pl.ds(lb, WA), :], o.at[pl.ds(g0, WA), :])

            def wb():
                pltpu.sync_copy(acc.at[pl.ds(lb + WA, WB), :],
                                o.at[pl.ds(g0 + WA, WB), :])
            if q < NSPLIT - 1:
                wb()
            else:
                @pl.when(si < 15)
                def _():
                    wb()

            @pl.when(si < 15)
            def _():
                pltpu.sync_copy(acc.at[pl.ds(lb + WA + WB, WC), :],
                                o.at[pl.ds(g0 + WA + WB, WC), :])

        @pl.when(ci == 0)
        def _():
            wout(om)

        @pl.when(ci == 1)
        def _():
            wout(oa)


def _deg2_call(dstm, dsta, ones128, zrows):
    dm, da = pl.kernel(
        _degs_body,
        out_type=[jax.ShapeDtypeStruct((NN, 128), f32)] * 2,
        mesh=_sc_mesh(),
        scratch_types=[
            pltpu.VMEM_SHARED((ACC_R, 128), f32),
            pltpu.VMEM((8, GW), i32),
            pltpu.VMEM((8, GW), i32),
            pltpu.VMEM((GW, 128), f32),
            pltpu.SemaphoreType.DMA,
        ],
    )(dstm, dsta, ones128, zrows)
    return dm[:, :1], da[:, :1]


def _rsq(deg):
    return jnp.where(deg > 0, lax.rsqrt(jnp.maximum(deg, 1e-12)), 0.0)


def _k0_body(x_ref, degm_ref, dega_ref, w1_ref, b1_ref, wk1_ref, wk2_ref,
             wr_ref, cst_ref, gm_ref, ga_ref, res_ref, dvm_ref, dva_ref):
    x = x_ref[...]
    h = jnp.maximum(
        jnp.dot(x, w1_ref[...], preferred_element_type=f32) + b1_ref[...], 0.0)
    dvm = _rsq(degm_ref[...])
    dva = _rsq(dega_ref[...])
    gm_ref[...] = jnp.dot(h, wk1_ref[...], preferred_element_type=f32) * dvm
    ga_ref[...] = jnp.dot(h, wk2_ref[...], preferred_element_type=f32) * dva
    res_ref[...] = jnp.dot(h, wr_ref[...], preferred_element_type=f32) \
        + cst_ref[0, 0]
    dvm_ref[...] = dvm
    dva_ref[...] = dva


def _kmid_body(am_ref, aa_ref, dvm_ref, dva_ref, resin_ref, b1_ref, b2_ref,
               wt1_ref, wb1_ref, wt2_ref, wb2_ref, wrt_ref, wrb_ref,
               gm_ref, ga_ref, resout_ref):
    dvm = dvm_ref[...]
    dva = dva_ref[...]
    r1 = am_ref[...] * dvm + b1_ref[...]
    r2 = aa_ref[...] * dva + b2_ref[...]
    gm_ref[...] = (jnp.dot(r1, wt1_ref[...], preferred_element_type=f32)
                   + jnp.dot(r2, wb1_ref[...], preferred_element_type=f32)) * dvm
    ga_ref[...] = (jnp.dot(r1, wt2_ref[...], preferred_element_type=f32)
                   + jnp.dot(r2, wb2_ref[...], preferred_element_type=f32)) * dva
    resout_ref[...] = resin_ref[...] \
        + jnp.dot(r1, wrt_ref[...], preferred_element_type=f32) \
        + jnp.dot(r2, wrb_ref[...], preferred_element_type=f32)


def _k3_body(am_ref, aa_ref, dvm_ref, dva_ref, resin_ref, b1_ref, b2_ref,
             wrt_ref, wrb_ref, out_ref):
    r1 = am_ref[...] * dvm_ref[...] + b1_ref[...]
    r2 = aa_ref[...] * dva_ref[...] + b2_ref[...]
    out_ref[...] = resin_ref[...] \
        + jnp.dot(r1, wrt_ref[...], preferred_element_type=f32) \
        + jnp.dot(r2, wrb_ref[...], preferred_element_type=f32)


def _rows(w):
    return pl.BlockSpec((BLK, w), lambda i: (i, 0))


def _full(shape):
    return pl.BlockSpec(shape, lambda i: (0, 0))


def _k0_call(x, degm, dega, w1, b1, wk1, wk2, wr, cst):
    return pl.pallas_call(
        _k0_body,
        grid=(GRID,),
        in_specs=[_rows(128), _rows(1), _rows(1), _full((128, 128)),
                  _full((1, 128)), _full((128, 128)), _full((128, 128)),
                  _full((128, 1)), _full((1, 1))],
        out_specs=[_rows(128), _rows(128), _rows(1), _rows(1), _rows(1)],
        out_shape=[jax.ShapeDtypeStruct((NN, 128), f32)] * 2
        + [jax.ShapeDtypeStruct((NN, 1), f32)] * 3,
    )(x, degm, dega, w1, b1, wk1, wk2, wr, cst)


def _kmid_call(am, aa, dvm, dva, resin, b1, b2, wt1, wb1, wt2, wb2, wrt, wrb):
    return pl.pallas_call(
        _kmid_body,
        grid=(GRID,),
        in_specs=[_rows(128)] * 2 + [_rows(1)] * 3
        + [_full((1, 128))] * 2 + [_full((128, 128))] * 4
        + [_full((128, 1))] * 2,
        out_specs=[_rows(128), _rows(128), _rows(1)],
        out_shape=[jax.ShapeDtypeStruct((NN, 128), f32)] * 2
        + [jax.ShapeDtypeStruct((NN, 1), f32)],
    )(am, aa, dvm, dva, resin, b1, b2, wt1, wb1, wt2, wb2, wrt, wrb)


def _k3_call(am, aa, dvm, dva, resin, b1, b2, wrt, wrb):
    return pl.pallas_call(
        _k3_body,
        grid=(GRID,),
        in_specs=[_rows(128)] * 2 + [_rows(1)] * 3
        + [_full((1, 128))] * 2 + [_full((128, 1))] * 2,
        out_specs=_rows(1),
        out_shape=jax.ShapeDtypeStruct((NN, 1), f32),
    )(am, aa, dvm, dva, resin, b1, b2, wrt, wrb)


def _pad_w(w):
    return jnp.zeros((128, 128), f32).at[:w.shape[0], :w.shape[1]].set(w)


def _pad_b(b):
    return jnp.zeros((1, 128), f32).at[0, :b.shape[0]].set(b)


def _pad_r(w, scale):
    return jnp.zeros((128, 1), f32).at[:w.shape[0], :].set(w * scale)


def _prep_edges(ei):
    pad = EP - EE
    srcp = jnp.concatenate([ei[0], jnp.zeros((pad,), i32)]).reshape(EPR, GW)
    dstp = jnp.concatenate([ei[1], jnp.full((pad,), NN, i32)]).reshape(EPR, GW)
    return srcp, dstp


def kernel(x, edge_index, edge_index_aux, W1, b1, Wk1_1, bk1_1, Wk1_2, bk1_2,
           Wk2_1, bk2_1, Wk2_2, bk2_2, Wk3_1, bk3_1, Wk3_2, bk3_2,
           Wr0, br0, Wr1, br1, Wr2, br2, Wr3, br3, w0, w1, w2, w3):
    srcm, dstm = _prep_edges(edge_index)
    srca, dsta = _prep_edges(edge_index_aux)
    ones128 = jnp.ones((GW, 128), f32)
    zrows = jnp.zeros((ZROWS, 128), f32)
    cst = (br0 * w0 + br1 * w1 + br2 * w2 + br3 * w3).reshape(1, 1)

    degm, dega = _deg2_call(dstm, dsta, ones128, zrows)

    gm, ga, res0, dvm, dva = _k0_call(
        x, degm, dega, _pad_w(W1), _pad_b(b1), _pad_w(Wk1_1), _pad_w(Wk1_2),
        _pad_r(Wr0, w0[0]), cst)

    a1m, a1a = _spmm_call(gm, ga, srcm, dstm, srca, dsta, zrows)
    gm, ga, res1 = _kmid_call(
        a1m, a1a, dvm, dva, res0, _pad_b(bk1_1), _pad_b(bk1_2),
        _pad_w(Wk2_1[:100]), _pad_w(Wk2_1[100:]),
        _pad_w(Wk2_2[:100]), _pad_w(Wk2_2[100:]),
        _pad_r(Wr1[:100], w1[0]), _pad_r(Wr1[100:], w1[0]))

    a2m, a2a = _spmm_call(gm, ga, srcm, dstm, srca, dsta, zrows)
    gm, ga, res2 = _kmid_call(
        a2m, a2a, dvm, dva, res1, _pad_b(bk2_1), _pad_b(bk2_2),
        _pad_w(Wk3_1[:100]), _pad_w(Wk3_1[100:]),
        _pad_w(Wk3_2[:100]), _pad_w(Wk3_2[100:]),
        _pad_r(Wr2[:100], w2[0]), _pad_r(Wr2[100:], w2[0]))

    a3m, a3a = _spmm_call(gm, ga, srcm, dstm, srca, dsta, zrows)
    return _k3_call(a3m, a3a, dvm, dva, res2, _pad_b(bk3_1), _pad_b(bk3_2),
                    _pad_r(Wr3[:100], w3[0]), _pad_r(Wr3[100:], w3[0]))


# speedup vs baseline: 1.3363x; 1.3363x over previous
import jax
import jax.numpy as jnp
from jax import lax
from jax.experimental import pallas as pl
from jax.experimental.pallas import tpu as pltpu
from jax.experimental.pallas import tpu_sc as plsc

NN = 50000
EE = 800000
EP = 802816
EPR = EP // 128
ROWS_T = EPR // 16
OUTER = ROWS_T // 8
NSPLIT = 4
QN = 12544
ACC_R = 12672
ZROWS = ACC_R // 16
WA = 488
WB = 176
WC = 128
DPT = 3136
BLK = 1000
GRID = NN // BLK

f32 = jnp.float32
i32 = jnp.int32


def _sc_mesh():
    return plsc.VectorSubcoreMesh(core_axis_name="c", subcore_axis_name="s")


def _spmm_body(gm, ga, srcm, dstm, srca, dsta, zrows, om, oa,
               acc, src_v, dst_v, dstq_v, rows0, sem0, sem1):
    ci = lax.axis_index("c")
    si = lax.axis_index("s")

    for q in range(NSPLIT):
        lb = pl.multiple_of(si * ZROWS, 8)
        pltpu.sync_copy(zrows, acc.at[pl.ds(lb, ZROWS), :])
        plsc.subcore_barrier()

        def edge_pass(src2d, dst2d, g):
            def body(ol, carry):
                off = pl.multiple_of(si * ROWS_T + ol * 8, 8)
                pltpu.sync_copy(src2d.at[pl.ds(off, 8)], src_v)
                pltpu.sync_copy(dst2d.at[pl.ds(off, 8)], dst_v)
                for j in range(8):
                    for v in range(8):
                        d = dst_v[j, pl.ds(16 * v, 16)]
                        ok = (d >= q * QN) & (d < q * QN + QN)
                        dstq_v[j, pl.ds(16 * v, 16)] = \
                            jnp.where(ok, d - q * QN, QN)
                for j in range(8):
                    pltpu.async_copy(g.at[src_v.at[j]], rows0, sem0).wait()
                    pltpu.async_copy(rows0, acc.at[dstq_v.at[j]], sem1,
                                     add=True).wait()
                return carry
            lax.fori_loop(0, OUTER, body, 0)

        @pl.when(ci == 0)
        def _():
            edge_pass(srcm, dstm, gm)

        @pl.when(ci == 1)
        def _():
            edge_pass(srca, dsta, ga)

        plsc.subcore_barrier()

        def wout(o):
            g0 = pl.multiple_of(q * QN + si * ZROWS, 8)
            pltpu.sync_copy(acc.at[pl.ds(lb, WA), :], o.at[pl.ds(g0, WA), :])

            def wb():
                pltpu.sync_copy(acc.at[pl.ds(lb + WA, WB), :],
                                o.at[pl.ds(g0 + WA, WB), :])
            if q < NSPLIT - 1:
                wb()
            else:
                @pl.when(si < 15)
                def _():
                    wb()

            @pl.when(si < 15)
            def _():
                pltpu.sync_copy(acc.at[pl.ds(lb + WA + WB, WC), :],
                                o.at[pl.ds(g0 + WA + WB, WC), :])

        @pl.when(ci == 0)
        def _():
            wout(om)

        @pl.when(ci == 1)
        def _():
            wout(oa)


def _spmm_call(gm, ga, srcm, dstm, srca, dsta, zrows):
    return pl.kernel(
        _spmm_body,
        out_type=[jax.ShapeDtypeStruct((NN, 128), f32)] * 2,
        mesh=_sc_mesh(),
        scratch_types=[
            pltpu.VMEM_SHARED((ACC_R, 128), f32),
            pltpu.VMEM((8, 128), i32),
            pltpu.VMEM((8, 128), i32),
            pltpu.VMEM((8, 128), i32),
            pltpu.VMEM((128, 128), f32),
            pltpu.SemaphoreType.DMA,
            pltpu.SemaphoreType.DMA,
        ],
    )(gm, ga, srcm, dstm, srca, dsta, zrows)


def _degs_body(dstm, dsta, ones128, zrows, om, oa, acc, dst_v, dstq_v, ones_v):
    ci = lax.axis_index("c")
    si = lax.axis_index("s")
    pltpu.sync_copy(ones128, ones_v)

    for q in range(NSPLIT):
        lb = pl.multiple_of(si * ZROWS, 8)
        pltpu.sync_copy(zrows, acc.at[pl.ds(lb, ZROWS), :])
        plsc.subcore_barrier()

        def edge_pass(dst2d):
            def body(ol, carry):
                off = pl.multiple_of(si * ROWS_T + ol * 8, 8)
                pltpu.sync_copy(dst2d.at[pl.ds(off, 8)], dst_v)
                for j in range(8):
                    for v in range(8):
                        d = dst_v[j, pl.ds(16 * v, 16)]
                        ok = (d >= q * QN) & (d < q * QN + QN)
                        dstq_v[j, pl.ds(16 * v, 16)] = \
                            jnp.where(ok, d - q * QN, QN)
                for j in range(8):
                    pltpu.sync_copy(ones_v, acc.at[dstq_v.at[j]], add=True)
                return carry
            lax.fori_loop(0, OUTER, body, 0)

        @pl.when(ci == 0)
        def _():
            edge_pass(dstm)

        @pl.when(ci == 1)
        def _():
            edge_pass(dsta)

        plsc.subcore_barrier()

        def wout(o):
            g0 = pl.multiple_of(q * QN + si * ZROWS, 8)
            pltpu.sync_copy(acc.at[pl.ds(lb, WA), :], o.at[pl.ds(g0, WA), :])

            def wb():
                pltpu.sync_copy(acc.at[pl.ds(lb + WA, WB), :],
                                o.at[pl.ds(g0 + WA, WB), :])
            if q < NSPLIT - 1:
                wb()
            else:
                @pl.when(si < 15)
                def _():
                    wb()

            @pl.when(si < 15)
            def _():
                pltpu.sync_copy(acc.at[pl.ds(lb + WA + WB, WC), :],
                                o.at[pl.ds(g0 + WA + WB, WC), :])

        @pl.when(ci == 0)
        def _():
            wout(om)

        @pl.when(ci == 1)
        def _():
            wout(oa)


def _deg2_call(dstm, dsta, ones128, zrows):
    dm, da = pl.kernel(
        _degs_body,
        out_type=[jax.ShapeDtypeStruct((NN, 128), f32)] * 2,
        mesh=_sc_mesh(),
        scratch_types=[
            pltpu.VMEM_SHARED((ACC_R, 128), f32),
            pltpu.VMEM((8, 128), i32),
            pltpu.VMEM((8, 128), i32),
            pltpu.VMEM((128, 128), f32),
        ],
    )(dstm, dsta, ones128, zrows)
    return dm[:, :1], da[:, :1]


def _rsq(deg):
    return jnp.where(deg > 0, lax.rsqrt(jnp.maximum(deg, 1e-12)), 0.0)


def _k0_body(x_ref, degm_ref, dega_ref, w1_ref, b1_ref, wk1_ref, wk2_ref,
             wr_ref, cst_ref, gm_ref, ga_ref, res_ref, dvm_ref, dva_ref):
    x = x_ref[...]
    h = jnp.maximum(
        jnp.dot(x, w1_ref[...], preferred_element_type=f32) + b1_ref[...], 0.0)
    dvm = _rsq(degm_ref[...])
    dva = _rsq(dega_ref[...])
    gm_ref[...] = jnp.dot(h, wk1_ref[...], preferred_element_type=f32) * dvm
    ga_ref[...] = jnp.dot(h, wk2_ref[...], preferred_element_type=f32) * dva
    res_ref[...] = jnp.dot(h, wr_ref[...], preferred_element_type=f32) \
        + cst_ref[0, 0]
    dvm_ref[...] = dvm
    dva_ref[...] = dva


def _kmid_body(am_ref, aa_ref, dvm_ref, dva_ref, resin_ref, b1_ref, b2_ref,
               wt1_ref, wb1_ref, wt2_ref, wb2_ref, wrt_ref, wrb_ref,
               gm_ref, ga_ref, resout_ref):
    dvm = dvm_ref[...]
    dva = dva_ref[...]
    r1 = am_ref[...] * dvm + b1_ref[...]
    r2 = aa_ref[...] * dva + b2_ref[...]
    gm_ref[...] = (jnp.dot(r1, wt1_ref[...], preferred_element_type=f32)
                   + jnp.dot(r2, wb1_ref[...], preferred_element_type=f32)) * dvm
    ga_ref[...] = (jnp.dot(r1, wt2_ref[...], preferred_element_type=f32)
                   + jnp.dot(r2, wb2_ref[...], preferred_element_type=f32)) * dva
    resout_ref[...] = resin_ref[...] \
        + jnp.dot(r1, wrt_ref[...], preferred_element_type=f32) \
        + jnp.dot(r2, wrb_ref[...], preferred_element_type=f32)


def _k3_body(am_ref, aa_ref, dvm_ref, dva_ref, resin_ref, b1_ref, b2_ref,
             wrt_ref, wrb_ref, out_ref):
    r1 = am_ref[...] * dvm_ref[...] + b1_ref[...]
    r2 = aa_ref[...] * dva_ref[...] + b2_ref[...]
    out_ref[...] = resin_ref[...] \
        + jnp.dot(r1, wrt_ref[...], preferred_element_type=f32) \
        + jnp.dot(r2, wrb_ref[...], preferred_element_type=f32)


def _rows(w):
    return pl.BlockSpec((BLK, w), lambda i: (i, 0))


def _full(shape):
    return pl.BlockSpec(shape, lambda i: (0, 0))


def _k0_call(x, degm, dega, w1, b1, wk1, wk2, wr, cst):
    return pl.pallas_call(
        _k0_body,
        grid=(GRID,),
        in_specs=[_rows(128), _rows(1), _rows(1), _full((128, 128)),
                  _full((1, 128)), _full((128, 128)), _full((128, 128)),
                  _full((128, 1)), _full((1, 1))],
        out_specs=[_rows(128), _rows(128), _rows(1), _rows(1), _rows(1)],
        out_shape=[jax.ShapeDtypeStruct((NN, 128), f32)] * 2
        + [jax.ShapeDtypeStruct((NN, 1), f32)] * 3,
    )(x, degm, dega, w1, b1, wk1, wk2, wr, cst)


def _kmid_call(am, aa, dvm, dva, resin, b1, b2, wt1, wb1, wt2, wb2, wrt, wrb):
    return pl.pallas_call(
        _kmid_body,
        grid=(GRID,),
        in_specs=[_rows(128)] * 2 + [_rows(1)] * 3
        + [_full((1, 128))] * 2 + [_full((128, 128))] * 4
        + [_full((128, 1))] * 2,
        out_specs=[_rows(128), _rows(128), _rows(1)],
        out_shape=[jax.ShapeDtypeStruct((NN, 128), f32)] * 2
        + [jax.ShapeDtypeStruct((NN, 1), f32)],
    )(am, aa, dvm, dva, resin, b1, b2, wt1, wb1, wt2, wb2, wrt, wrb)


def _k3_call(am, aa, dvm, dva, resin, b1, b2, wrt, wrb):
    return pl.pallas_call(
        _k3_body,
        grid=(GRID,),
        in_specs=[_rows(128)] * 2 + [_rows(1)] * 3
        + [_full((1, 128))] * 2 + [_full((128, 1))] * 2,
        out_specs=_rows(1),
        out_shape=jax.ShapeDtypeStruct((NN, 1), f32),
    )(am, aa, dvm, dva, resin, b1, b2, wrt, wrb)


def _pad_w(w):
    return jnp.zeros((128, 128), f32).at[:w.shape[0], :w.shape[1]].set(w)


def _pad_b(b):
    return jnp.zeros((1, 128), f32).at[0, :b.shape[0]].set(b)


def _pad_r(w, scale):
    return jnp.zeros((128, 1), f32).at[:w.shape[0], :].set(w * scale)


def _prep_edges(ei):
    pad = EP - EE
    srcp = jnp.concatenate([ei[0], jnp.zeros((pad,), i32)]).reshape(EPR, 128)
    dstp = jnp.concatenate([ei[1], jnp.full((pad,), NN, i32)]).reshape(EPR, 128)
    return srcp, dstp


def kernel(x, edge_index, edge_index_aux, W1, b1, Wk1_1, bk1_1, Wk1_2, bk1_2,
           Wk2_1, bk2_1, Wk2_2, bk2_2, Wk3_1, bk3_1, Wk3_2, bk3_2,
           Wr0, br0, Wr1, br1, Wr2, br2, Wr3, br3, w0, w1, w2, w3):
    srcm, dstm = _prep_edges(edge_index)
    srca, dsta = _prep_edges(edge_index_aux)
    ones128 = jnp.ones((128, 128), f32)
    zrows = jnp.zeros((ZROWS, 128), f32)
    cst = (br0 * w0 + br1 * w1 + br2 * w2 + br3 * w3).reshape(1, 1)

    degm, dega = _deg2_call(dstm, dsta, ones128, zrows)

    gm, ga, res0, dvm, dva = _k0_call(
        x, degm, dega, _pad_w(W1), _pad_b(b1), _pad_w(Wk1_1), _pad_w(Wk1_2),
        _pad_r(Wr0, w0[0]), cst)

    a1m, a1a = _spmm_call(gm, ga, srcm, dstm, srca, dsta, zrows)
    gm, ga, res1 = _kmid_call(
        a1m, a1a, dvm, dva, res0, _pad_b(bk1_1), _pad_b(bk1_2),
        _pad_w(Wk2_1[:100]), _pad_w(Wk2_1[100:]),
        _pad_w(Wk2_2[:100]), _pad_w(Wk2_2[100:]),
        _pad_r(Wr1[:100], w1[0]), _pad_r(Wr1[100:], w1[0]))

    a2m, a2a = _spmm_call(gm, ga, srcm, dstm, srca, dsta, zrows)
    gm, ga, res2 = _kmid_call(
        a2m, a2a, dvm, dva, res1, _pad_b(bk2_1), _pad_b(bk2_2),
        _pad_w(Wk3_1[:100]), _pad_w(Wk3_1[100:]),
        _pad_w(Wk3_2[:100]), _pad_w(Wk3_2[100:]),
        _pad_r(Wr2[:100], w2[0]), _pad_r(Wr2[100:], w2[0]))

    a3m, a3a = _spmm_call(gm, ga, srcm, dstm, srca, dsta, zrows)
    return _k3_call(a3m, a3a, dvm, dva, res2, _pad_b(bk3_1), _pad_b(bk3_2),
                    _pad_r(Wr3[:100], w3[0]), _pad_r(Wr3[100:], w3[0]))
